# baseline (speedup 1.0000x reference)
import functools

import jax
import jax.numpy as jnp
from jax.experimental import pallas as pl
from jax.experimental.pallas import tpu as pltpu


def _gcn_body(x_ref, w_ref, b_ref, pw_ref, adj_ref, out_ref, h_ref):
    i = pl.program_id(0)

    @pl.when(i == 0)
    def _():
        h = (
            jnp.dot(x_ref[...], w_ref[...].T, preferred_element_type=jnp.float32)
            + b_ref[...]
        )
        h_ref[...] = h.astype(jnp.bfloat16)

    a = adj_ref[...].astype(jnp.bfloat16)
    y = jnp.dot(a, h_ref[...], preferred_element_type=jnp.float32)
    pw = pw_ref[0, 0]
    out_ref[...] = jnp.where(y >= 0, y, pw * y)


@functools.partial(jax.jit, static_argnames=("bm",))
def _gcn(x2, adj, W, b2, pw2, bm):
    n, f_in = x2.shape
    f_hid = W.shape[0]
    grid = (n // bm,)
    return pl.pallas_call(
        _gcn_body,
        grid=grid,
        in_specs=[
            pl.BlockSpec((n, f_in), lambda i: (0, 0)),
            pl.BlockSpec((f_hid, f_in), lambda i: (0, 0)),
            pl.BlockSpec((1, f_hid), lambda i: (0, 0)),
            pl.BlockSpec((1, 1), lambda i: (0, 0)),
            pl.BlockSpec((bm, n), lambda i: (i, 0)),
        ],
        out_specs=pl.BlockSpec((bm, f_hid), lambda i: (i, 0)),
        out_shape=jax.ShapeDtypeStruct((n, f_hid), jnp.float32),
        scratch_shapes=[pltpu.VMEM((n, f_hid), jnp.bfloat16)],
        compiler_params=pltpu.CompilerParams(
            dimension_semantics=("arbitrary",),
        ),
    )(x2, W, b2, pw2, adj)


def kernel(x, adj, W, b, prelu_w):
    n = adj.shape[0]
    x2 = jnp.reshape(x, (n, x.shape[-1]))
    b2 = jnp.reshape(b, (1, -1))
    pw2 = jnp.reshape(prelu_w, (1, 1))
    y = _gcn(x2, adj, W, b2, pw2, bm=200)
    return jnp.expand_dims(y, axis=0)


# speedup vs baseline: 1.0138x; 1.0138x over previous
import functools

import jax
import jax.numpy as jnp
from jax.experimental import pallas as pl
from jax.experimental.pallas import tpu as pltpu


def _gcn_body(x_ref, w_ref, b_ref, pw_ref, adj_ref, out_ref, h_ref):
    i = pl.program_id(0)

    @pl.when(i == 0)
    def _():
        h = (
            jnp.dot(x_ref[...], w_ref[...].T, preferred_element_type=jnp.float32)
            + b_ref[...]
        )
        h_ref[...] = h.astype(jnp.bfloat16)

    a = adj_ref[...].astype(jnp.bfloat16)
    y = jnp.dot(a, h_ref[...], preferred_element_type=jnp.float32)
    pw = pw_ref[0, 0]
    out_ref[...] = jnp.where(y >= 0, y, pw * y)


@functools.partial(jax.jit, static_argnames=("bm",))
def _gcn(x2, adj, W, b2, pw2, bm):
    n, f_in = x2.shape
    f_hid = W.shape[0]
    grid = (n // bm,)
    return pl.pallas_call(
        _gcn_body,
        grid=grid,
        in_specs=[
            pl.BlockSpec((n, f_in), lambda i: (0, 0)),
            pl.BlockSpec((f_hid, f_in), lambda i: (0, 0)),
            pl.BlockSpec((1, f_hid), lambda i: (0, 0)),
            pl.BlockSpec((1, 1), lambda i: (0, 0)),
            pl.BlockSpec((bm, n), lambda i: (i, 0)),
        ],
        out_specs=pl.BlockSpec((bm, f_hid), lambda i: (i, 0)),
        out_shape=jax.ShapeDtypeStruct((n, f_hid), jnp.float32),
        scratch_shapes=[pltpu.VMEM((n, f_hid), jnp.bfloat16)],
        compiler_params=pltpu.CompilerParams(
            dimension_semantics=("arbitrary",),
            vmem_limit_bytes=128 * 1024 * 1024,
        ),
    )(x2, W, b2, pw2, adj)


def kernel(x, adj, W, b, prelu_w):
    n = adj.shape[0]
    x2 = jnp.reshape(x, (n, x.shape[-1]))
    b2 = jnp.reshape(b, (1, -1))
    pw2 = jnp.reshape(prelu_w, (1, 1))
    y = _gcn(x2, adj, W, b2, pw2, bm=400)
    return jnp.expand_dims(y, axis=0)
